# baseline (speedup 1.0000x reference)
import dataclasses
import functools

import jax
import jax.numpy as jnp
from jax import lax
from jax.experimental import pallas as pl
from jax.experimental.pallas import tpu as pltpu
from jax.experimental.pallas import tpu_sc as plsc

_DIM = 2048
_NE = 64
_K = 8
_BT = 2048
_S = 8
_SB = _BT // _S
_T = 16384
_NW = 32
_PER = _T // _NW


def _logits_block(*refs):
    x_refs = refs[:_S]
    w_ref = refs[_S]
    out_ref = refs[_S + 1]
    w = w_ref[...]
    for sb in range(_S):
        lg = jax.lax.dot_general(
            x_refs[sb][...],
            w,
            (((1,), (1,)), ((), ())),
            preferred_element_type=jnp.float32,
        )
        out_ref[pl.ds(sb * _SB, _SB), :] = lg


def _tc_logits(x, weight):
    t = x.shape[0]
    grid = (t // _BT,)
    return pl.pallas_call(
        _logits_block,
        grid=grid,
        in_specs=[
            pl.BlockSpec(
                (_SB, _DIM), functools.partial(lambda s, i: (i * _S + s, 0), s)
            )
            for s in range(_S)
        ]
        + [
            pl.BlockSpec((_NE, _DIM), lambda i: (0, 0)),
        ],
        out_specs=pl.BlockSpec((_BT, _NE), lambda i: (i, 0)),
        out_shape=jax.ShapeDtypeStruct((t, _NE), jnp.float32),
    )(*([x] * _S), weight)


def _sc_topk(scores):
    mesh = plsc.VectorSubcoreMesh(core_axis_name="c", subcore_axis_name="s")
    cp = pltpu.CompilerParams()
    if "needs_layout_passes" in pltpu.CompilerParams.__dataclass_fields__:
        cp = dataclasses.replace(cp, needs_layout_passes=False)

    @functools.partial(
        pl.kernel,
        compiler_params=cp,
        out_type=[
            jax.ShapeDtypeStruct((_K, _T), jnp.float32),
            jax.ShapeDtypeStruct((_K, _T), jnp.int32),
        ],
        mesh=mesh,
        scratch_types=[
            pltpu.VMEM((_PER, _NE), jnp.float32),
            pltpu.VMEM((_K, _PER), jnp.float32),
            pltpu.VMEM((_K, _PER), jnp.int32),
            pltpu.SemaphoreType.DMA,
        ],
    )
    def k(sc_hbm, vals_hbm, idx_hbm, chunk, vbuf, ibuf, sem):
        wid = lax.axis_index("s") * 2 + lax.axis_index("c")
        base = wid * _PER
        pltpu.async_copy(sc_hbm.at[pl.ds(base, _PER)], chunk, sem).wait()
        iota = lax.iota(jnp.int32, 16)
        rank_mask = iota < _K

        def merge(a, b):
            ak, av = a
            bk, bv = b
            rbk = lax.rev(bk, (0,))
            rbv = lax.rev(bv, (0,))
            take = ak >= rbk
            mk = jnp.where(take, ak, rbk)
            mv = jnp.where(take, av, rbv)
            return plsc.sort_key_val(mk, mv, descending=True)

        @pl.loop(0, _PER)
        def _(t):
            groups = []
            for g in range(4):
                kg = chunk[t, pl.ds(g * 16, 16)]
                groups.append(plsc.sort_key_val(kg, iota + g * 16, descending=True))
            mk, mv = merge(merge(groups[0], groups[1]), merge(groups[2], groups[3]))
            sig = 1.0 / (1.0 + jnp.exp(-mk))
            sig = jnp.where(rank_mask, sig, jnp.float32(0.0))
            tot = jnp.sum(sig)
            nrm = sig / tot
            col = jnp.full((16,), t, jnp.int32)
            plsc.store_scatter(vbuf, [iota, col], nrm, mask=rank_mask)
            plsc.store_scatter(ibuf, [iota, col], mv, mask=rank_mask)

        pltpu.async_copy(vbuf, vals_hbm.at[:, pl.ds(base, _PER)], sem).wait()
        pltpu.async_copy(ibuf, idx_hbm.at[:, pl.ds(base, _PER)], sem).wait()

    return k(scores)


@jax.jit
def kernel(x, weight):
    scores = _tc_logits(x, weight)
    vals, idx = _sc_topk(scores)
    return vals.T, idx.T


# speedup vs baseline: 2.2405x; 2.2405x over previous
import functools

import jax
import jax.numpy as jnp
from jax.experimental import pallas as pl

_DIM = 2048
_NE = 64
_K = 8
_BT = 2048
_S = 8
_SB = _BT // _S


def _gate_block(*refs):
    x_refs = refs[:_S]
    w_ref = refs[_S]
    vals_ref, idx_ref = refs[_S + 1], refs[_S + 2]
    w = w_ref[...]
    rev_f = ((_NE - 1) - jax.lax.broadcasted_iota(jnp.int32, (_NE, _SB), 0)).astype(
        jnp.float32
    )
    neg_inf = jnp.float32(-jnp.inf)
    for sb in range(_S):
        lt = jax.lax.dot_general(
            w,
            x_refs[sb][...],
            (((1,), (1,)), ((), ())),
            preferred_element_type=jnp.float32,
        )
        tops = []
        ridxs = []
        for _ in range(_K):
            m = jnp.max(lt, axis=0, keepdims=True)
            r = jnp.max(
                jnp.where(lt == m, rev_f, jnp.float32(-1.0)), axis=0, keepdims=True
            )
            lt = jnp.where(rev_f == r, neg_inf, lt)
            tops.append(m)
            ridxs.append(r)
        top_val = jnp.concatenate(tops, axis=0)
        top_idx = (_NE - 1) - jnp.concatenate(ridxs, axis=0).astype(jnp.int32)
        s = jax.nn.sigmoid(top_val)
        s = s / jnp.sum(s, axis=0, keepdims=True)
        cols = pl.ds(sb * _SB, _SB)
        vals_ref[:, cols] = s
        idx_ref[:, cols] = top_idx


@jax.jit
def kernel(x, weight):
    t = x.shape[0]
    grid = (t // _BT,)
    vals, idx = pl.pallas_call(
        _gate_block,
        grid=grid,
        in_specs=[
            pl.BlockSpec(
                (_SB, _DIM), functools.partial(lambda s, i: (i * _S + s, 0), s)
            )
            for s in range(_S)
        ]
        + [
            pl.BlockSpec((_NE, _DIM), lambda i: (0, 0)),
        ],
        out_specs=[
            pl.BlockSpec((_K, _BT), lambda i: (0, i)),
            pl.BlockSpec((_K, _BT), lambda i: (0, i)),
        ],
        out_shape=[
            jax.ShapeDtypeStruct((_K, t), jnp.float32),
            jax.ShapeDtypeStruct((_K, t), jnp.int32),
        ],
    )(*([x] * _S), weight)
    return vals.T, idx.T
